# Initial kernel scaffold; baseline (speedup 1.0000x reference)
import jax
import jax.numpy as jnp
from jax.experimental import pallas as pl


def kernel(x, Wg, W1, b1, W2, b2):
    raise NotImplementedError("write your pallas kernel here")


# speedup vs baseline: 1.7271x; 1.7271x over previous
import functools

import jax
import jax.numpy as jnp
from jax.experimental import pallas as pl
from jax.experimental.pallas import tpu as pltpu

BT = 256
FT = 512


def _gate_kernel(x_ref, wg_ref, i0_ref, i1_ref, w0_ref, w1_ref):
    x = x_ref[...]
    wg = wg_ref[...]
    logits = jax.lax.dot_general(
        x, wg, (((1,), (1,)), ((), ())), preferred_element_type=jnp.float32
    )
    T, E = logits.shape
    iota = jax.lax.broadcasted_iota(jnp.int32, (T, E), 1)
    m0 = jnp.max(logits, axis=1, keepdims=True)
    i0 = jnp.min(jnp.where(logits == m0, iota, E), axis=1, keepdims=True)
    masked = jnp.where(iota == i0, -jnp.inf, logits)
    m1 = jnp.max(masked, axis=1, keepdims=True)
    i1 = jnp.min(jnp.where(masked == m1, iota, E), axis=1, keepdims=True)
    z = jnp.exp(m1 - m0)
    w0 = 1.0 / (1.0 + z)
    i0_ref[...] = i0
    i1_ref[...] = i1
    w0_ref[...] = w0
    w1_ref[...] = 1.0 - w0


def _moe_ffn_kernel(tok_s, off_s, x_ref, ws_ref, w1_ref, b1_ref, w2_ref,
                    b2_ref, out_ref, xe_ref, y_ref, *, nf):
    e = pl.program_id(0)
    ft = pl.program_id(1)
    start = off_s[e]
    cnt = off_s[e + 1] - start
    nch = (cnt + BT - 1) // BT

    @pl.when((e == 0) & (ft == 0))
    def _():
        out_ref[...] = jnp.zeros_like(out_ref)

    @pl.when(ft == 0)
    def _():
        def gbody(i, c):
            t = tok_s[start + i]
            xe_ref[pl.ds(i, 1), :] = x_ref[pl.ds(t, 1), :]
            return c
        jax.lax.fori_loop(0, cnt, gbody, 0)

    w1 = w1_ref[0]
    w2 = w2_ref[0]
    b1 = b1_ref[0, 0]

    def cbody(c, carry):
        xs = xe_ref[pl.ds(c * BT, BT), :]
        h = jnp.dot(xs, w1, preferred_element_type=jnp.float32) + b1
        h = 0.5 * h * (1.0 + jax.lax.erf(h * 0.7071067811865476))
        yp = jnp.dot(h, w2, preferred_element_type=jnp.float32)
        prev = jnp.where(ft == 0, 0.0, y_ref[pl.ds(c * BT, BT), :])
        y_ref[pl.ds(c * BT, BT), :] = prev + yp
        return carry
    jax.lax.fori_loop(0, nch, cbody, 0)

    @pl.when(ft == nf - 1)
    def _():
        b2 = b2_ref[0]

        def sbody(i, c):
            p = start + i
            t = tok_s[p]
            w = ws_ref[pl.ds(p, 1), :]
            row = (y_ref[pl.ds(i, 1), :] + b2) * w
            out_ref[pl.ds(t, 1), :] += row
            return c
        jax.lax.fori_loop(0, cnt, sbody, 0)


def kernel(x, Wg, W1, b1, W2, b2):
    B, S, D = x.shape
    E, _, F = W1.shape
    T = B * S
    K = 2
    TK = T * K
    nf = F // FT
    x_flat = x.reshape(T, D)

    i0, i1, w0, w1v = pl.pallas_call(
        _gate_kernel,
        out_shape=[
            jax.ShapeDtypeStruct((T, 1), jnp.int32),
            jax.ShapeDtypeStruct((T, 1), jnp.int32),
            jax.ShapeDtypeStruct((T, 1), jnp.float32),
            jax.ShapeDtypeStruct((T, 1), jnp.float32),
        ],
    )(x_flat, Wg)

    e_flat = jnp.concatenate([i0, i1], axis=1).reshape(TK)
    w_flat = jnp.concatenate([w0, w1v], axis=1).reshape(TK)
    perm = jnp.argsort(e_flat)
    tok_sorted = (perm // K).astype(jnp.int32)
    w_sorted = w_flat[perm].reshape(TK, 1)
    counts = jnp.sum(e_flat[:, None] == jnp.arange(E)[None, :], axis=0)
    off = jnp.concatenate(
        [jnp.zeros((1,), jnp.int32), jnp.cumsum(counts).astype(jnp.int32)]
    )

    grid = (E, nf)
    out = pl.pallas_call(
        functools.partial(_moe_ffn_kernel, nf=nf),
        grid_spec=pltpu.PrefetchScalarGridSpec(
            num_scalar_prefetch=2,
            grid=grid,
            in_specs=[
                pl.BlockSpec((T, D), lambda e, f, *_: (0, 0)),
                pl.BlockSpec((TK, 1), lambda e, f, *_: (0, 0)),
                pl.BlockSpec((1, D, FT), lambda e, f, *_: (e, 0, f)),
                pl.BlockSpec((1, 1, 1, FT), lambda e, f, *_: (e, f, 0, 0)),
                pl.BlockSpec((1, FT, D), lambda e, f, *_: (e, f, 0)),
                pl.BlockSpec((1, 1, D), lambda e, f, *_: (e, 0, 0)),
            ],
            out_specs=pl.BlockSpec((T, D), lambda e, f, *_: (0, 0)),
            scratch_shapes=[
                pltpu.VMEM((T, D), jnp.float32),
                pltpu.VMEM((T, D), jnp.float32),
            ],
        ),
        out_shape=jax.ShapeDtypeStruct((T, D), jnp.float32),
        compiler_params=pltpu.CompilerParams(
            dimension_semantics=("arbitrary", "arbitrary"),
        ),
    )(tok_sorted, off, x_flat, w_sorted, W1,
      b1.reshape(E, nf, 1, FT), W2, b2.reshape(E, 1, D))

    return out.reshape(B, S, D)
